# baseline (speedup 1.0000x reference)
import functools

import jax
import jax.numpy as jnp
from jax import lax
from jax.experimental import pallas as pl
from jax.experimental.pallas import tpu as pltpu
from jax.experimental.pallas import tpu_sc as plsc

N_VOCAB = 100000
D_MODEL = 1024
BATCH = 4
SEQ = 4096
B_TOTAL = BATCH * SEQ

_info = plsc.get_sparse_core_info()
NC = _info.num_cores
NS = _info.num_subcores
NW = NC * NS
B_PER_W = B_TOTAL // NW
CHUNK = 8
N_CHUNKS = B_PER_W // CHUNK
NBUF = 14
AHEAD = 7
LAG = 3


def _emb_kernel(table_hbm, idx_hbm, out_hbm, idx_v, rows_v, gsem, ssem):
    wid = lax.axis_index("s") * NC + lax.axis_index("c")
    base = wid * B_PER_W

    w_per_b = SEQ // B_PER_W
    pltpu.sync_copy(
        idx_hbm.at[wid // w_per_b, pl.ds((wid % w_per_b) * B_PER_W, B_PER_W)],
        idx_v,
    )

    def gather(j, buf):
        return pltpu.async_copy(
            table_hbm.at[idx_v.at[pl.ds(j * CHUNK, CHUNK)]],
            rows_v.at[buf],
            gsem.at[buf],
        )

    def scatter(j, buf):
        return pltpu.async_copy(
            rows_v.at[buf], out_hbm.at[pl.ds(base + j * CHUNK, CHUNK)], ssem.at[buf]
        )

    def wait_gather(j, buf):
        pltpu.make_async_copy(
            table_hbm.at[idx_v.at[pl.ds(j * CHUNK, CHUNK)]],
            rows_v.at[buf],
            gsem.at[buf],
        ).wait()

    def wait_scatter(j, buf):
        pltpu.make_async_copy(
            rows_v.at[buf], out_hbm.at[pl.ds(base + j * CHUNK, CHUNK)], ssem.at[buf]
        ).wait()

    for j in range(AHEAD):
        gather(j, j)
    waited = -1
    for j in range(N_CHUNKS):
        buf = j % NBUF
        wait_gather(j, buf)
        scatter(j, buf)
        if j + AHEAD < N_CHUNKS:
            if j >= LAG:
                wait_scatter(j - LAG, (j - LAG) % NBUF)
                waited = j - LAG
            gather(j + AHEAD, (j + AHEAD) % NBUF)
    for j in range(waited + 1, N_CHUNKS):
        wait_scatter(j, j % NBUF)


@jax.jit
def _embed(x, w_ei):
    mesh = plsc.VectorSubcoreMesh(core_axis_name="c", subcore_axis_name="s")
    run = functools.partial(
        pl.kernel,
        mesh=mesh,
        out_type=jax.ShapeDtypeStruct((B_TOTAL, D_MODEL), jnp.float32),
        scratch_types=[
            pltpu.VMEM((B_PER_W,), jnp.int32),
            pltpu.VMEM((NBUF, CHUNK, D_MODEL), jnp.float32),
            pltpu.SemaphoreType.DMA((NBUF,)),
            pltpu.SemaphoreType.DMA((NBUF,)),
        ],
    )(_emb_kernel)
    return run(w_ei, x)


def kernel(x, w_ei):
    out = _embed(x.astype(jnp.int32), w_ei.astype(jnp.float32))
    return out.reshape(BATCH, SEQ, D_MODEL)


# speedup vs baseline: 1.0345x; 1.0345x over previous
import functools

import jax
import jax.numpy as jnp
from jax import lax
from jax.experimental import pallas as pl
from jax.experimental.pallas import tpu as pltpu
from jax.experimental.pallas import tpu_sc as plsc

N_VOCAB = 100000
D_MODEL = 1024
BATCH = 4
SEQ = 4096
B_TOTAL = BATCH * SEQ

_info = plsc.get_sparse_core_info()
NC = _info.num_cores
NS = _info.num_subcores
NW = NC * NS
B_PER_W = B_TOTAL // NW
CHUNK = 16
N_CHUNKS = B_PER_W // CHUNK
NBUF = 7
AHEAD = 5
LAG = 2


def _emb_kernel(table_hbm, idx_hbm, out_hbm, idx_v, rows_v, gsem, ssem):
    wid = lax.axis_index("s") * NC + lax.axis_index("c")
    base = wid * B_PER_W

    w_per_b = SEQ // B_PER_W
    pltpu.sync_copy(
        idx_hbm.at[wid // w_per_b, pl.ds((wid % w_per_b) * B_PER_W, B_PER_W)],
        idx_v,
    )

    def gather(j, buf):
        return pltpu.async_copy(
            table_hbm.at[idx_v.at[pl.ds(j * CHUNK, CHUNK)]],
            rows_v.at[buf],
            gsem.at[buf],
        )

    def scatter(j, buf):
        return pltpu.async_copy(
            rows_v.at[buf], out_hbm.at[pl.ds(base + j * CHUNK, CHUNK)], ssem.at[buf]
        )

    def wait_gather(j, buf):
        pltpu.make_async_copy(
            table_hbm.at[idx_v.at[pl.ds(j * CHUNK, CHUNK)]],
            rows_v.at[buf],
            gsem.at[buf],
        ).wait()

    def wait_scatter(j, buf):
        pltpu.make_async_copy(
            rows_v.at[buf], out_hbm.at[pl.ds(base + j * CHUNK, CHUNK)], ssem.at[buf]
        ).wait()

    for j in range(AHEAD):
        gather(j, j)
    waited = -1
    for j in range(N_CHUNKS):
        buf = j % NBUF
        wait_gather(j, buf)
        scatter(j, buf)
        if j + AHEAD < N_CHUNKS:
            if j >= LAG:
                wait_scatter(j - LAG, (j - LAG) % NBUF)
                waited = j - LAG
            gather(j + AHEAD, (j + AHEAD) % NBUF)
    for j in range(waited + 1, N_CHUNKS):
        wait_scatter(j, j % NBUF)


@jax.jit
def _embed(x, w_ei):
    mesh = plsc.VectorSubcoreMesh(core_axis_name="c", subcore_axis_name="s")
    run = functools.partial(
        pl.kernel,
        mesh=mesh,
        out_type=jax.ShapeDtypeStruct((B_TOTAL, D_MODEL), jnp.float32),
        scratch_types=[
            pltpu.VMEM((B_PER_W,), jnp.int32),
            pltpu.VMEM((NBUF, CHUNK, D_MODEL), jnp.float32),
            pltpu.SemaphoreType.DMA((NBUF,)),
            pltpu.SemaphoreType.DMA((NBUF,)),
        ],
    )(_emb_kernel)
    return run(w_ei, x)


def kernel(x, w_ei):
    out = _embed(x.astype(jnp.int32), w_ei.astype(jnp.float32))
    return out.reshape(BATCH, SEQ, D_MODEL)
